# baseline (speedup 1.0000x reference)
import jax
import jax.numpy as jnp
from jax import lax
from jax.experimental import pallas as pl
from jax.experimental.pallas import tpu as pltpu
from jax.experimental.pallas import tpu_sc as plsc

N = 200000
C = 32
K = 27
EPS = 1e-5

NC = 2
NS = 16
NW = NC * NS

R = 64
CH = N // R
CPW_MAX = -(-CH // NW)
REM = CH - (CPW_MAX - 1) * NW

MB = 800
EB = 8000


def _split_store_dma(y, i, o_refs, stage, sem):
    for k in range(K):
        stage[k] = y[:, k * C:(k + 1) * C]
    copies = [
        pltpu.async_copy(stage.at[k], o_refs[k].at[pl.ds(i * MB, MB)], sem)
        for k in range(K)
    ]
    for cp in copies:
        cp.wait()


def _matmul_split_body(x_ref, w_ref, *o_refs_scratch):
    o_refs = o_refs_scratch[:K]
    stage, sem = o_refs_scratch[K:]
    i = pl.program_id(0)
    y = jnp.dot(x_ref[...], w_ref[...], preferred_element_type=jnp.float32)
    _split_store_dma(y, i, o_refs, stage, sem)


def _tc_matmul_split(x, w_mat):
    return pl.pallas_call(
        _matmul_split_body,
        grid=(N // MB,),
        in_specs=[
            pl.BlockSpec((MB, C), lambda i: (i, 0)),
            pl.BlockSpec((C, K * C), lambda i: (0, 0)),
        ],
        out_specs=[pl.BlockSpec(memory_space=pltpu.MemorySpace.HBM)] * K,
        out_shape=[jax.ShapeDtypeStruct((N, C), jnp.float32)] * K,
        scratch_shapes=[
            pltpu.VMEM((K, MB, C), jnp.float32),
            pltpu.SemaphoreType.DMA,
        ],
    )(x, w_mat)


def _gather_sum_body(*refs):
    tables = refs[:K]
    idx_hbm = refs[K]
    out_hbm = refs[K + 1]
    stats_hbm = refs[K + 2]
    idx_v, buf, acc, stats_v, sem = refs[K + 3:]

    wid = lax.axis_index("s") * NC + lax.axis_index("c")
    n_chunks = CPW_MAX - 1 + jnp.where(wid < REM, 1, 0)

    def chunk(j, carry):
        s0, s1, q0, q1 = carry
        c = j * NW + wid
        pltpu.sync_copy(idx_hbm.at[c], idx_v)
        copies = [
            pltpu.async_copy(tables[k].at[idx_v.at[k]],
                             buf.at[pl.ds(k * R, R)], sem)
            for k in range(K)
        ]
        for cp in copies:
            cp.wait()

        def site(n, carry2):
            s0, s1, q0, q1 = carry2
            a0 = buf[n, pl.ds(0, 16)]
            a1 = buf[n, pl.ds(16, 16)]
            for k in range(1, K):
                a0 = a0 + buf[k * R + n, pl.ds(0, 16)]
                a1 = a1 + buf[k * R + n, pl.ds(16, 16)]
            acc[n, pl.ds(0, 16)] = a0
            acc[n, pl.ds(16, 16)] = a1
            return (s0 + a0, s1 + a1, q0 + a0 * a0, q1 + a1 * a1)

        carry = lax.fori_loop(0, R, site, (s0, s1, q0, q1), unroll=False)
        pltpu.sync_copy(acc, out_hbm.at[pl.ds(c * R, R)])
        return carry

    z = jnp.zeros((16,), jnp.float32)
    s0, s1, q0, q1 = lax.fori_loop(0, n_chunks, chunk, (z, z, z, z),
                                   unroll=False)
    stats_v[pl.ds(0, 16)] = s0
    stats_v[pl.ds(16, 16)] = s1
    stats_v[pl.ds(32, 16)] = q0
    stats_v[pl.ds(48, 16)] = q1
    pltpu.sync_copy(stats_v, stats_hbm.at[wid])


def _sc_gather_sum(tables, idx3):
    mesh = plsc.VectorSubcoreMesh(core_axis_name="c", subcore_axis_name="s",
                                  num_cores=NC, num_subcores=NS)
    out, stats = pl.kernel(
        _gather_sum_body,
        out_type=[
            jax.ShapeDtypeStruct((N, C), jnp.float32),
            jax.ShapeDtypeStruct((NW, 2 * C), jnp.float32),
        ],
        mesh=mesh,
        scratch_types=[
            pltpu.VMEM((K, R), jnp.int32),
            pltpu.VMEM((K * R, C), jnp.float32),
            pltpu.VMEM((R, C), jnp.float32),
            pltpu.VMEM((2 * C,), jnp.float32),
            pltpu.SemaphoreType.DMA,
        ],
        compiler_params=pltpu.CompilerParams(use_tc_tiling_on_sc=False),
    )(*tables, idx3)
    return out, stats


def _bn_scale_shift(stats_ref, g_ref, b_ref):
    s = jnp.sum(stats_ref[...], axis=0)
    mean = s[:C] * (1.0 / N)
    var = s[C:] * (1.0 / N) - mean * mean
    scale = g_ref[...] * lax.rsqrt(var + EPS)
    shift = b_ref[...] - mean * scale
    return scale, shift


def _bn_relu_matmul_body(h_ref, stats_ref, g_ref, b_ref, w_ref,
                         *o_refs_scratch):
    o_refs = o_refs_scratch[:K]
    stage, sem = o_refs_scratch[K:]
    i = pl.program_id(0)
    scale, shift = _bn_scale_shift(stats_ref, g_ref, b_ref)
    z = jnp.maximum(h_ref[...] * scale[None, :] + shift[None, :], 0.0)
    y = jnp.dot(z, w_ref[...], preferred_element_type=jnp.float32)
    _split_store_dma(y, i, o_refs, stage, sem)


def _tc_bn_relu_matmul(h, stats, gamma, beta, w_mat):
    return pl.pallas_call(
        _bn_relu_matmul_body,
        grid=(N // MB,),
        in_specs=[
            pl.BlockSpec((MB, C), lambda i: (i, 0)),
            pl.BlockSpec((NW, 2 * C), lambda i: (0, 0)),
            pl.BlockSpec((C,), lambda i: (0,)),
            pl.BlockSpec((C,), lambda i: (0,)),
            pl.BlockSpec((C, K * C), lambda i: (0, 0)),
        ],
        out_specs=[pl.BlockSpec(memory_space=pltpu.MemorySpace.HBM)] * K,
        out_shape=[jax.ShapeDtypeStruct((N, C), jnp.float32)] * K,
        scratch_shapes=[
            pltpu.VMEM((K, MB, C), jnp.float32),
            pltpu.SemaphoreType.DMA,
        ],
    )(h, stats, gamma, beta, w_mat)


def _bn_res_relu_body(h_ref, stats_ref, g_ref, b_ref, x_ref, o_ref):
    scale, shift = _bn_scale_shift(stats_ref, g_ref, b_ref)
    o_ref[...] = jnp.maximum(
        h_ref[...] * scale[None, :] + shift[None, :] + x_ref[...], 0.0)


def _tc_bn_res_relu(h, stats, gamma, beta, x):
    return pl.pallas_call(
        _bn_res_relu_body,
        grid=(N // EB,),
        in_specs=[
            pl.BlockSpec((EB, C), lambda i: (i, 0)),
            pl.BlockSpec((NW, 2 * C), lambda i: (0, 0)),
            pl.BlockSpec((C,), lambda i: (0,)),
            pl.BlockSpec((C,), lambda i: (0,)),
            pl.BlockSpec((EB, C), lambda i: (i, 0)),
        ],
        out_specs=pl.BlockSpec((EB, C), lambda i: (i, 0)),
        out_shape=jax.ShapeDtypeStruct((N, C), jnp.float32),
    )(h, stats, gamma, beta, x)


@jax.jit
def kernel(x, nbr_idx, W1, gamma1, beta1, W2, gamma2, beta2):
    w1m = W1.transpose(1, 0, 2).reshape(C, K * C)
    w2m = W2.transpose(1, 0, 2).reshape(C, K * C)

    idx3 = nbr_idx.reshape(CH, R, K).transpose(0, 2, 1)

    y1 = _tc_matmul_split(x, w1m)
    h1, st1 = _sc_gather_sum(y1, idx3)
    y2 = _tc_bn_relu_matmul(h1, st1, gamma1, beta1, w2m)
    h2, st2 = _sc_gather_sum(y2, idx3)
    return _tc_bn_res_relu(h2, st2, gamma2, beta2, x)


# speedup vs baseline: 3.3571x; 3.3571x over previous
import jax
import jax.numpy as jnp
from jax import lax
from jax.experimental import pallas as pl
from jax.experimental.pallas import tpu as pltpu
from jax.experimental.pallas import tpu_sc as plsc

N = 200000
C = 32
K = 27
KP = 28
EPS = 1e-5

NC = 2
NS = 16
NW = NC * NS

R = 64
GP, GSZ = 18, 96
assert GP * GSZ == R * K
CH = N // R
CPW_MAX = -(-CH // NW)
REM = CH - (CPW_MAX - 1) * NW

MB = 800
MBF = MB // 4
EB = 8000
Q = N // 4


def _slab_write(o_ref, p, yp):
    for j in range(KP * C // 128):
        o_ref[p, :, j, :] = yp[:, j * 128:(j + 1) * 128]


def _matmul_slab_body(x_ref, w_ref, o_ref):
    for p in range(4):
        xp = x_ref[:, p * C:(p + 1) * C]
        yp = jnp.dot(xp, w_ref[...], preferred_element_type=jnp.float32)
        _slab_write(o_ref, p, yp)


def _tc_matmul_slab(x4, w_mat):
    return pl.pallas_call(
        _matmul_slab_body,
        grid=(N // MB,),
        in_specs=[
            pl.BlockSpec((MBF, 128), lambda i: (i, 0)),
            pl.BlockSpec((C, KP * C), lambda i: (0, 0)),
        ],
        out_specs=pl.BlockSpec((4, MBF, 8, 128), lambda i: (0, i, 0, 0)),
        out_shape=jax.ShapeDtypeStruct((4, Q, 8, 128), jnp.float32),
    )(x4, w_mat)


def _gather_sum_body(table, idx_hbm, out_hbm, stats_hbm,
                     idx_v, buf, acc, stats_v, sem):
    wid = lax.axis_index("s") * NC + lax.axis_index("c")
    n_chunks = CPW_MAX - 1 + jnp.where(wid < REM, 1, 0)

    def chunk(j, carry):
        s0, s1, q0, q1 = carry
        c = j * NW + wid
        pltpu.sync_copy(idx_hbm.at[c], idx_v)
        copies = [
            pltpu.async_copy(table.at[idx_v.at[g]],
                             buf.at[pl.ds(g * GSZ, GSZ)], sem)
            for g in range(GP)
        ]
        for cp in copies:
            cp.wait()

        def site(n, carry2):
            s0, s1, q0, q1 = carry2
            a0 = buf[n * K, pl.ds(0, 16)]
            a1 = buf[n * K, pl.ds(16, 16)]
            for k in range(1, K):
                a0 = a0 + buf[n * K + k, pl.ds(0, 16)]
                a1 = a1 + buf[n * K + k, pl.ds(16, 16)]
            acc[n, pl.ds(0, 16)] = a0
            acc[n, pl.ds(16, 16)] = a1
            return (s0 + a0, s1 + a1, q0 + a0 * a0, q1 + a1 * a1)

        carry = lax.fori_loop(0, R, site, (s0, s1, q0, q1), unroll=False)
        pltpu.sync_copy(acc, out_hbm.at[pl.ds(c * R, R)])
        return carry

    z = jnp.zeros((16,), jnp.float32)
    s0, s1, q0, q1 = lax.fori_loop(0, n_chunks, chunk, (z, z, z, z),
                                   unroll=False)
    stats_v[pl.ds(0, 16)] = s0
    stats_v[pl.ds(16, 16)] = s1
    stats_v[pl.ds(32, 16)] = q0
    stats_v[pl.ds(48, 16)] = q1
    pltpu.sync_copy(stats_v, stats_hbm.at[wid])


def _sc_gather_sum(table_flat, idx3):
    mesh = plsc.VectorSubcoreMesh(core_axis_name="c", subcore_axis_name="s",
                                  num_cores=NC, num_subcores=NS)
    out, stats = pl.kernel(
        _gather_sum_body,
        out_type=[
            jax.ShapeDtypeStruct((N, C), jnp.float32),
            jax.ShapeDtypeStruct((NW, 2 * C), jnp.float32),
        ],
        mesh=mesh,
        scratch_types=[
            pltpu.VMEM((GP, GSZ), jnp.int32),
            pltpu.VMEM((R * K, C), jnp.float32),
            pltpu.VMEM((R, C), jnp.float32),
            pltpu.VMEM((2 * C,), jnp.float32),
            pltpu.SemaphoreType.DMA,
        ],
        compiler_params=pltpu.CompilerParams(use_tc_tiling_on_sc=False),
    )(table_flat, idx3)
    return out, stats


def _bn_scale_shift(stats_ref, g_ref, b_ref):
    s = jnp.sum(stats_ref[...], axis=0)
    mean = s[:C] * (1.0 / N)
    var = s[C:] * (1.0 / N) - mean * mean
    scale = g_ref[...] * lax.rsqrt(var + EPS)
    shift = b_ref[...] - mean * scale
    scale4 = jnp.concatenate([scale] * 4)
    shift4 = jnp.concatenate([shift] * 4)
    return scale4, shift4


def _bn_relu_matmul_body(h_ref, stats_ref, g_ref, b_ref, w_ref, o_ref):
    scale4, shift4 = _bn_scale_shift(stats_ref, g_ref, b_ref)
    z = jnp.maximum(h_ref[...] * scale4[None, :] + shift4[None, :], 0.0)
    for p in range(4):
        zp = z[:, p * C:(p + 1) * C]
        yp = jnp.dot(zp, w_ref[...], preferred_element_type=jnp.float32)
        _slab_write(o_ref, p, yp)


def _tc_bn_relu_matmul(h4, stats, gamma, beta, w_mat):
    return pl.pallas_call(
        _bn_relu_matmul_body,
        grid=(N // MB,),
        in_specs=[
            pl.BlockSpec((MBF, 128), lambda i: (i, 0)),
            pl.BlockSpec((NW, 2 * C), lambda i: (0, 0)),
            pl.BlockSpec((C,), lambda i: (0,)),
            pl.BlockSpec((C,), lambda i: (0,)),
            pl.BlockSpec((C, KP * C), lambda i: (0, 0)),
        ],
        out_specs=pl.BlockSpec((4, MBF, 8, 128), lambda i: (0, i, 0, 0)),
        out_shape=jax.ShapeDtypeStruct((4, Q, 8, 128), jnp.float32),
    )(h4, stats, gamma, beta, w_mat)


def _bn_res_relu_body(h_ref, stats_ref, g_ref, b_ref, x_ref, o_ref):
    scale4, shift4 = _bn_scale_shift(stats_ref, g_ref, b_ref)
    o_ref[...] = jnp.maximum(
        h_ref[...] * scale4[None, :] + shift4[None, :] + x_ref[...], 0.0)


def _tc_bn_res_relu(h4, stats, gamma, beta, x4):
    return pl.pallas_call(
        _bn_res_relu_body,
        grid=(N // EB,),
        in_specs=[
            pl.BlockSpec((EB // 4, 128), lambda i: (i, 0)),
            pl.BlockSpec((NW, 2 * C), lambda i: (0, 0)),
            pl.BlockSpec((C,), lambda i: (0,)),
            pl.BlockSpec((C,), lambda i: (0,)),
            pl.BlockSpec((EB // 4, 128), lambda i: (i, 0)),
        ],
        out_specs=pl.BlockSpec((EB // 4, 128), lambda i: (i, 0)),
        out_shape=jax.ShapeDtypeStruct((Q, 128), jnp.float32),
    )(h4, stats, gamma, beta, x4)


def _w_mat(W):
    wp = jnp.pad(W, ((0, KP - K), (0, 0), (0, 0)))
    return wp.transpose(1, 0, 2).reshape(C, KP * C)


@jax.jit
def kernel(x, nbr_idx, W1, gamma1, beta1, W2, gamma2, beta2):
    w1m = _w_mat(W1)
    w2m = _w_mat(W2)
    x4 = x.reshape(Q, 128)

    m = nbr_idx
    fi = (m % 4) * (8 * N) + (m // 4) * C + jnp.arange(K, dtype=jnp.int32)
    idx3 = fi.reshape(CH, GP, GSZ)

    t1 = _tc_matmul_slab(x4, w1m)
    h1, st1 = _sc_gather_sum(t1.reshape(32 * N, C), idx3)
    t2 = _tc_bn_relu_matmul(h1.reshape(Q, 128), st1, gamma1, beta1, w2m)
    h2, st2 = _sc_gather_sum(t2.reshape(32 * N, C), idx3)
    o4 = _tc_bn_res_relu(h2.reshape(Q, 128), st2, gamma2, beta2, x4)
    return o4.reshape(N, C)


# speedup vs baseline: 4.4051x; 1.3122x over previous
import jax
import jax.numpy as jnp
from jax import lax
from jax.experimental import pallas as pl
from jax.experimental.pallas import tpu as pltpu
from jax.experimental.pallas import tpu_sc as plsc

N = 200000
C = 32
K = 27
KP = 28
EPS = 1e-5

NC = 2
NS = 16
NW = NC * NS

R = 64
GP, GSZ = 18, 96
assert GP * GSZ == R * K
CH = N // R
CPW_MAX = -(-CH // NW)
REM = CH - (CPW_MAX - 1) * NW

MB = 800
MBF = MB // 4
EB = 8000
Q = N // 4


def _slab_write(o_ref, p, yp):
    for j in range(KP * C // 128):
        o_ref[p, :, j, :] = yp[:, j * 128:(j + 1) * 128]


def _matmul_slab_body(x_ref, w_ref, o_ref):
    for p in range(4):
        xp = x_ref[:, p * C:(p + 1) * C]
        yp = jnp.dot(xp, w_ref[...], preferred_element_type=jnp.float32)
        _slab_write(o_ref, p, yp)


def _tc_matmul_slab(x4, w_mat):
    return pl.pallas_call(
        _matmul_slab_body,
        grid=(N // MB,),
        in_specs=[
            pl.BlockSpec((MBF, 128), lambda i: (i, 0)),
            pl.BlockSpec((C, KP * C), lambda i: (0, 0)),
        ],
        out_specs=pl.BlockSpec((4, MBF, 8, 128), lambda i: (0, i, 0, 0)),
        out_shape=jax.ShapeDtypeStruct((4, Q, 8, 128), jnp.float32),
    )(x4, w_mat)


def _gather_sum_body(table, idx_hbm, out_hbm, stats_hbm,
                     idx_v, buf, acc, stats_v, sems):
    wid = lax.axis_index("s") * NC + lax.axis_index("c")
    isems = sems[:4]
    gsems = sems[4:]
    BUFR = R * K

    def valid(j):
        return j * NW + wid < CH

    def fire_idx(j, islot):
        @pl.when(valid(j))
        def _():
            pltpu.async_copy(idx_hbm.at[j * NW + wid],
                             idx_v.at[pl.ds(islot * GP, GP)], isems[islot])

    def fire_gathers(j, islot, bslot):
        @pl.when(valid(j))
        def _():
            pltpu.make_async_copy(idx_hbm.at[0],
                                  idx_v.at[pl.ds(islot * GP, GP)],
                                  isems[islot]).wait()
            for g in range(GP):
                pltpu.async_copy(table.at[idx_v.at[islot * GP + g]],
                                 buf.at[pl.ds(bslot * BUFR + g * GSZ, GSZ)],
                                 gsems[bslot])

    def accum(j, bslot, carry):
        ok = valid(j)

        @pl.when(ok)
        def _():
            for g in range(GP):
                pltpu.make_async_copy(
                    table.at[idx_v.at[g]],
                    buf.at[pl.ds(g * GSZ, GSZ)], gsems[bslot]).wait()

        boff = bslot * BUFR

        def site(n, carry2):
            s0, s1, q0, q1 = carry2
            a0 = buf[boff + n * K, pl.ds(0, 16)]
            a1 = buf[boff + n * K, pl.ds(16, 16)]
            for k in range(1, K):
                a0 = a0 + buf[boff + n * K + k, pl.ds(0, 16)]
                a1 = a1 + buf[boff + n * K + k, pl.ds(16, 16)]
            acc[n, pl.ds(0, 16)] = a0
            acc[n, pl.ds(16, 16)] = a1
            b0 = jnp.where(ok, a0, 0.0)
            b1 = jnp.where(ok, a1, 0.0)
            return (s0 + b0, s1 + b1, q0 + b0 * b0, q1 + b1 * b1)

        carry = lax.fori_loop(0, R, site, carry, unroll=False)

        @pl.when(ok)
        def _():
            pltpu.sync_copy(acc, out_hbm.at[pl.ds((j * NW + wid) * R, R)])
        return carry

    fire_idx(0, 0)
    fire_idx(1, 1)
    fire_idx(2, 2)
    fire_gathers(0, 0, 0)

    NQUAD = -(-CPW_MAX // 4)

    def quad(t, carry):
        j = 4 * t
        for i in range(4):
            fire_gathers(j + i + 1, (i + 1) % 4, (i + 1) % 2)
            fire_idx(j + i + 3, (i + 3) % 4)
            carry = accum(j + i, i % 2, carry)
        return carry

    z = jnp.zeros((16,), jnp.float32)
    s0, s1, q0, q1 = lax.fori_loop(0, NQUAD, quad, (z, z, z, z),
                                   unroll=False)
    stats_v[pl.ds(0, 16)] = s0
    stats_v[pl.ds(16, 16)] = s1
    stats_v[pl.ds(32, 16)] = q0
    stats_v[pl.ds(48, 16)] = q1
    pltpu.sync_copy(stats_v, stats_hbm.at[wid])


def _sc_gather_sum(table_flat, idx3):
    mesh = plsc.VectorSubcoreMesh(core_axis_name="c", subcore_axis_name="s",
                                  num_cores=NC, num_subcores=NS)
    out, stats = pl.kernel(
        _gather_sum_body,
        out_type=[
            jax.ShapeDtypeStruct((N, C), jnp.float32),
            jax.ShapeDtypeStruct((NW, 2 * C), jnp.float32),
        ],
        mesh=mesh,
        scratch_types=[
            pltpu.VMEM((4 * GP, GSZ), jnp.int32),
            pltpu.VMEM((2 * R * K, C), jnp.float32),
            pltpu.VMEM((R, C), jnp.float32),
            pltpu.VMEM((2 * C,), jnp.float32),
            (pltpu.SemaphoreType.DMA,) * 6,
        ],
        compiler_params=pltpu.CompilerParams(use_tc_tiling_on_sc=False),
    )(table_flat, idx3)
    return out, stats


def _bn_scale_shift(stats_ref, g_ref, b_ref):
    s = jnp.sum(stats_ref[...], axis=0)
    mean = s[:C] * (1.0 / N)
    var = s[C:] * (1.0 / N) - mean * mean
    scale = g_ref[...] * lax.rsqrt(var + EPS)
    shift = b_ref[...] - mean * scale
    scale4 = jnp.concatenate([scale] * 4)
    shift4 = jnp.concatenate([shift] * 4)
    return scale4, shift4


def _bn_relu_matmul_body(h_ref, stats_ref, g_ref, b_ref, w_ref, o_ref):
    scale4, shift4 = _bn_scale_shift(stats_ref, g_ref, b_ref)
    z = jnp.maximum(h_ref[...] * scale4[None, :] + shift4[None, :], 0.0)
    for p in range(4):
        zp = z[:, p * C:(p + 1) * C]
        yp = jnp.dot(zp, w_ref[...], preferred_element_type=jnp.float32)
        _slab_write(o_ref, p, yp)


def _tc_bn_relu_matmul(h4, stats, gamma, beta, w_mat):
    return pl.pallas_call(
        _bn_relu_matmul_body,
        grid=(N // MB,),
        in_specs=[
            pl.BlockSpec((MBF, 128), lambda i: (i, 0)),
            pl.BlockSpec((NW, 2 * C), lambda i: (0, 0)),
            pl.BlockSpec((C,), lambda i: (0,)),
            pl.BlockSpec((C,), lambda i: (0,)),
            pl.BlockSpec((C, KP * C), lambda i: (0, 0)),
        ],
        out_specs=pl.BlockSpec((4, MBF, 8, 128), lambda i: (0, i, 0, 0)),
        out_shape=jax.ShapeDtypeStruct((4, Q, 8, 128), jnp.float32),
    )(h4, stats, gamma, beta, w_mat)


def _bn_res_relu_body(h_ref, stats_ref, g_ref, b_ref, x_ref, o_ref):
    scale4, shift4 = _bn_scale_shift(stats_ref, g_ref, b_ref)
    o_ref[...] = jnp.maximum(
        h_ref[...] * scale4[None, :] + shift4[None, :] + x_ref[...], 0.0)


def _tc_bn_res_relu(h4, stats, gamma, beta, x4):
    return pl.pallas_call(
        _bn_res_relu_body,
        grid=(N // EB,),
        in_specs=[
            pl.BlockSpec((EB // 4, 128), lambda i: (i, 0)),
            pl.BlockSpec((NW, 2 * C), lambda i: (0, 0)),
            pl.BlockSpec((C,), lambda i: (0,)),
            pl.BlockSpec((C,), lambda i: (0,)),
            pl.BlockSpec((EB // 4, 128), lambda i: (i, 0)),
        ],
        out_specs=pl.BlockSpec((EB // 4, 128), lambda i: (i, 0)),
        out_shape=jax.ShapeDtypeStruct((Q, 128), jnp.float32),
    )(h4, stats, gamma, beta, x4)


def _w_mat(W):
    wp = jnp.pad(W, ((0, KP - K), (0, 0), (0, 0)))
    return wp.transpose(1, 0, 2).reshape(C, KP * C)


@jax.jit
def kernel(x, nbr_idx, W1, gamma1, beta1, W2, gamma2, beta2):
    w1m = _w_mat(W1)
    w2m = _w_mat(W2)
    x4 = x.reshape(Q, 128)

    m = nbr_idx
    fi = (m % 4) * (8 * N) + (m // 4) * C + jnp.arange(K, dtype=jnp.int32)
    idx3 = fi.reshape(CH, GP, GSZ)

    t1 = _tc_matmul_slab(x4, w1m)
    h1, st1 = _sc_gather_sum(t1.reshape(32 * N, C), idx3)
    t2 = _tc_bn_relu_matmul(h1.reshape(Q, 128), st1, gamma1, beta1, w2m)
    h2, st2 = _sc_gather_sum(t2.reshape(32 * N, C), idx3)
    o4 = _tc_bn_res_relu(h2.reshape(Q, 128), st2, gamma2, beta2, x4)
    return o4.reshape(N, C)
